# Initial kernel scaffold; baseline (speedup 1.0000x reference)
import jax
import jax.numpy as jnp
from jax.experimental import pallas as pl


def kernel(nf, edge_index, edge_type, We0_0, be0_0, We0_1, be0_1, Wn0, bn0, We1_0, be1_0, We1_1, be1_1, Wn1, bn1, We2_0, be2_0, We2_1, be2_1, Wn2, bn2):
    raise NotImplementedError("write your pallas kernel here")


# speedup vs baseline: 8.2118x; 8.2118x over previous
import functools

import jax
import jax.numpy as jnp
from jax import lax
from jax.experimental import pallas as pl
from jax.experimental.pallas import tpu as pltpu
from jax.experimental.pallas import tpu_sc as plsc

_NC = 2
_NS = 16
_BLK = 80
_BN = 1000


def _table_body(x_ref, wa_ref, wb_ref, be_ref, a_ref, b_ref):
    x = x_ref[...]
    a_ref[0] = jnp.dot(x, wa_ref[0], preferred_element_type=jnp.float32)
    a_ref[1] = jnp.dot(x, wa_ref[1], preferred_element_type=jnp.float32)
    b_ref[0] = jnp.dot(x, wb_ref[0], preferred_element_type=jnp.float32) + be_ref[0:1, :]
    b_ref[1] = jnp.dot(x, wb_ref[1], preferred_element_type=jnp.float32) + be_ref[1:2, :]


def _table_call(h, wa, wb, be, n, din):
    grid = (n // _BN,)
    return pl.pallas_call(
        _table_body,
        grid=grid,
        in_specs=[
            pl.BlockSpec((_BN, din), lambda i: (i, 0)),
            pl.BlockSpec((2, din, 32), lambda i: (0, 0, 0)),
            pl.BlockSpec((2, din, 32), lambda i: (0, 0, 0)),
            pl.BlockSpec((2, 32), lambda i: (0, 0)),
        ],
        out_specs=[
            pl.BlockSpec((2, _BN, 32), lambda i: (0, i, 0)),
            pl.BlockSpec((2, _BN, 32), lambda i: (0, i, 0)),
        ],
        out_shape=[
            jax.ShapeDtypeStruct((2, n, 32), jnp.float32),
            jax.ShapeDtypeStruct((2, n, 32), jnp.float32),
        ],
    )(h, wa, wb, be)


def _post_body(s00, s01, s10, s11, c00, c01, c10, c11,
               h_ref, wa_ref, wb_ref, wc_ref, bn_ref, o_ref):
    c0 = jnp.maximum(c00[:, 0:1] + c10[:, 0:1], 1.0)
    c1 = jnp.maximum(c01[:, 0:1] + c11[:, 0:1], 1.0)
    m0 = (s00[...] + s10[...]) / c0
    m1 = (s01[...] + s11[...]) / c1
    o = (jnp.dot(m0, wa_ref[...], preferred_element_type=jnp.float32)
         + jnp.dot(m1, wb_ref[...], preferred_element_type=jnp.float32)
         + jnp.dot(h_ref[...], wc_ref[...], preferred_element_type=jnp.float32)
         + bn_ref[...])
    o_ref[...] = jnp.maximum(o, 0.0)


def _post_call(S, C, h, wn, bn, n, din, dout):
    nb = n // _BN
    s_specs = [pl.BlockSpec((_BN, 32), lambda i, k=k: (i + k * nb, 0))
               for k in range(4)]
    c_specs = [pl.BlockSpec((_BN, 16), lambda i, k=k: (i + k * nb, 0))
               for k in range(4)]
    return pl.pallas_call(
        _post_body,
        grid=(nb,),
        in_specs=s_specs + c_specs + [
            pl.BlockSpec((_BN, din), lambda i: (i, 0)),
            pl.BlockSpec((32, dout), lambda i: (0, 0)),
            pl.BlockSpec((32, dout), lambda i: (0, 0)),
            pl.BlockSpec((din, dout), lambda i: (0, 0)),
            pl.BlockSpec((1, dout), lambda i: (0, 0)),
        ],
        out_specs=pl.BlockSpec((_BN, dout), lambda i: (i, 0)),
        out_shape=jax.ShapeDtypeStruct((n, dout), jnp.float32),
    )(S, S, S, S, C, C, C, C, h,
      wn[:32], wn[32:64], wn[64:], bn.reshape(1, dout))


def _edge_body(with_counts, n2, E, *refs):
    per_w = E // (_NC * _NS)
    nb = per_w // _BLK
    rpt_lo = (n2 // _NS) // 8 * 8
    rpt_hi = n2 - rpt_lo * (_NS - 1)
    if with_counts:
        (ia_hbm, ib_hbm, a_hbm, b_hbm, s_hbm, c_hbm,
         ia_v, ib_v, ar, br, zb, ones_v, zb16, acc, cacc, sem_a, sem_b) = refs
    else:
        (ia_hbm, ib_hbm, a_hbm, b_hbm, s_hbm,
         ia_v, ib_v, ar, br, zb, acc, sem_a, sem_b) = refs

    cid = lax.axis_index("c")
    sid = lax.axis_index("s")
    wid = sid * _NC + cid

    zero16 = jnp.zeros((16,), jnp.float32)
    last = sid == _NS - 1

    def zrow(r, c):
        zb[r, pl.ds(0, 16)] = zero16
        zb[r, pl.ds(16, 16)] = zero16
        return c
    lax.fori_loop(0, rpt_hi, zrow, 0)

    @pl.when(jnp.logical_not(last))
    def _():
        pltpu.sync_copy(zb.at[pl.ds(0, rpt_lo)],
                        acc.at[pl.ds(sid * rpt_lo, rpt_lo)])

    @pl.when(last)
    def _():
        pltpu.sync_copy(zb, acc.at[pl.ds((_NS - 1) * rpt_lo, rpt_hi)])

    if with_counts:
        one0 = jnp.where(lax.iota(jnp.int32, 16) < 1,
                         jnp.full((16,), 1.0, jnp.float32),
                         jnp.zeros((16,), jnp.float32))

        def orow(r, c):
            ones_v[r, pl.ds(0, 16)] = one0
            return c
        lax.fori_loop(0, _BLK, orow, 0)

        def z16row(r, c):
            zb16[r, pl.ds(0, 16)] = zero16
            return c
        lax.fori_loop(0, rpt_hi, z16row, 0)

        @pl.when(jnp.logical_not(last))
        def _():
            pltpu.sync_copy(zb16.at[pl.ds(0, rpt_lo)],
                            cacc.at[pl.ds(sid * rpt_lo, rpt_lo)])

        @pl.when(last)
        def _():
            pltpu.sync_copy(zb16, cacc.at[pl.ds((_NS - 1) * rpt_lo, rpt_hi)])

    plsc.subcore_barrier()

    def blk(b, c):
        off = wid * per_w + b * _BLK
        pltpu.sync_copy(ia_hbm.at[pl.ds(off, _BLK)], ia_v)
        pltpu.sync_copy(ib_hbm.at[pl.ds(off, _BLK)], ib_v)
        da = pltpu.async_copy(a_hbm.at[ia_v], ar, sem_a)
        db = pltpu.async_copy(b_hbm.at[ib_v], br, sem_b)
        da.wait()
        db.wait()

        def rw(r, cc):
            for hh in range(2):
                sl = pl.ds(16 * hh, 16)
                br[r, sl] = jnp.maximum(ar[r, sl] + br[r, sl], 0.0)
            return cc
        lax.fori_loop(0, _BLK, rw, 0)

        pltpu.sync_copy(br, acc.at[ib_v], add=True)
        if with_counts:
            pltpu.sync_copy(ones_v, cacc.at[ib_v], add=True)
        return c
    lax.fori_loop(0, nb, blk, 0)

    plsc.subcore_barrier()
    roff = cid * n2 + sid * rpt_lo

    @pl.when(jnp.logical_not(last))
    def _():
        pltpu.sync_copy(acc.at[pl.ds(sid * rpt_lo, rpt_lo)],
                        s_hbm.at[pl.ds(roff, rpt_lo)])
        if with_counts:
            pltpu.sync_copy(cacc.at[pl.ds(sid * rpt_lo, rpt_lo)],
                            c_hbm.at[pl.ds(roff, rpt_lo)])

    @pl.when(last)
    def _():
        pltpu.sync_copy(acc.at[pl.ds((_NS - 1) * rpt_lo, rpt_hi)],
                        s_hbm.at[pl.ds(roff, rpt_hi)])
        if with_counts:
            pltpu.sync_copy(cacc.at[pl.ds((_NS - 1) * rpt_lo, rpt_hi)],
                            c_hbm.at[pl.ds(roff, rpt_hi)])


@functools.lru_cache(maxsize=None)
def _edge_call(n2, E, with_counts):
    rpt_lo = (n2 // _NS) // 8 * 8
    rpt = n2 - rpt_lo * (_NS - 1)
    mesh = plsc.VectorSubcoreMesh(core_axis_name="c", subcore_axis_name="s",
                                  num_cores=_NC, num_subcores=_NS)
    out_type = [jax.ShapeDtypeStruct((_NC * n2, 32), jnp.float32)]
    if with_counts:
        out_type.append(jax.ShapeDtypeStruct((_NC * n2, 16), jnp.float32))
    scratch = [
        pltpu.VMEM((_BLK,), jnp.int32),
        pltpu.VMEM((_BLK,), jnp.int32),
        pltpu.VMEM((_BLK, 32), jnp.float32),
        pltpu.VMEM((_BLK, 32), jnp.float32),
        pltpu.VMEM((rpt, 32), jnp.float32),
    ]
    if with_counts:
        scratch += [
            pltpu.VMEM((_BLK, 16), jnp.float32),
            pltpu.VMEM((rpt, 16), jnp.float32),
        ]
    scratch += [pltpu.VMEM_SHARED((n2, 32), jnp.float32)]
    if with_counts:
        scratch += [pltpu.VMEM_SHARED((n2, 16), jnp.float32)]
    scratch += [pltpu.SemaphoreType.DMA, pltpu.SemaphoreType.DMA]
    return pl.kernel(
        functools.partial(_edge_body, with_counts, n2, E),
        mesh=mesh,
        compiler_params=pltpu.CompilerParams(use_tc_tiling_on_sc=False),
        out_type=out_type,
        scratch_types=scratch,
    )


def kernel(nf, edge_index, edge_type,
           We0_0, be0_0, We0_1, be0_1, Wn0, bn0,
           We1_0, be1_0, We1_1, be1_1, Wn1, bn1,
           We2_0, be2_0, We2_1, be2_1, Wn2, bn2):
    n = nf.shape[0]
    E = edge_type.shape[0]
    n2 = 2 * n
    src, dst = edge_index[0], edge_index[1]
    ia = edge_type * n + src
    ib = edge_type * n + dst

    params = [
        (We0_0, be0_0, We0_1, be0_1, Wn0, bn0),
        (We1_0, be1_0, We1_1, be1_1, Wn1, bn1),
        (We2_0, be2_0, We2_1, be2_1, Wn2, bn2),
    ]
    h = nf
    C = None
    for l, (we0, be0, we1, be1, wn, bnv) in enumerate(params):
        din = h.shape[1]
        wa = jnp.stack([we0[:din], we1[:din]])
        wb = jnp.stack([we0[din:], we1[din:]])
        be = jnp.stack([be0, be1])
        A, B = _table_call(h, wa, wb, be, n, din)
        A2 = A.reshape(n2, 32)
        B2 = B.reshape(n2, 32)
        if l == 0:
            S, C = _edge_call(n2, E, True)(ia, ib, A2, B2)
        else:
            (S,) = _edge_call(n2, E, False)(ia, ib, A2, B2)
        dout = wn.shape[1]
        h = _post_call(S, C, h, wn, bnv, n, din, dout)
    return h
